# baseline (speedup 1.0000x reference)
import jax
import jax.numpy as jnp
from jax import lax
from jax.experimental import pallas as pl
from jax.experimental.pallas import tpu as pltpu
from jax.experimental.pallas import tpu_sc as plsc

NUM_CORES = 2
NUM_SUBCORES = 16
LANES = 16
NUM_WORKERS = NUM_CORES * NUM_SUBCORES

BATCH = 16384
EMBED_DIM = 32
CHUNK = BATCH // NUM_WORKERS
IDX_BLK = 128
NBLK = CHUNK // IDX_BLK
GROUPS = CHUNK // LANES
VECS = IDX_BLK // LANES

SCRATCH_TYPES = [
    pltpu.VMEM((NBLK, IDX_BLK), jnp.int32),
    pltpu.VMEM((NBLK, IDX_BLK), jnp.int32),
    pltpu.VMEM((NBLK, IDX_BLK), jnp.int32),
    pltpu.VMEM((NBLK, IDX_BLK), jnp.int32),
    pltpu.VMEM((NBLK, IDX_BLK, EMBED_DIM), jnp.float32),
    pltpu.VMEM((NBLK, IDX_BLK, EMBED_DIM), jnp.float32),
    pltpu.VMEM((NBLK, IDX_BLK, LANES), jnp.float32),
    pltpu.VMEM((NBLK, IDX_BLK, LANES), jnp.float32),
    pltpu.VMEM((EMBED_DIM,), jnp.float32),
    pltpu.VMEM((LANES,), jnp.float32),
    pltpu.VMEM((CHUNK, LANES), jnp.float32),
    pltpu.VMEM((CHUNK,), jnp.float32),
    pltpu.SemaphoreType.DMA,
]


def _rec_body(uid_hbm, mid_hbm, ut_hbm, mt_hbm, ubt_hbm, mbt_hbm, w_hbm,
              outb_hbm, out_hbm, uid_v, mid_v, ubi_v, mbi_v, u_v, m_v, ub_v,
              mb_v, w_v, outb_v, q_v, o_v, sem):
    wid = lax.axis_index("s") * NUM_CORES + lax.axis_index("c")
    base = wid * CHUNK

    pltpu.sync_copy(uid_hbm.at[pl.ds(wid * NBLK, NBLK)], uid_v)
    pltpu.sync_copy(mid_hbm.at[pl.ds(wid * NBLK, NBLK)], mid_v)
    pltpu.sync_copy(w_hbm, w_v)
    pltpu.sync_copy(outb_hbm, outb_v)

    copies = []
    for j in range(NBLK):
        copies.append(pltpu.async_copy(ut_hbm.at[uid_v.at[j]], u_v.at[j], sem))
        copies.append(pltpu.async_copy(mt_hbm.at[mid_v.at[j]], m_v.at[j], sem))

    for j in range(NBLK):
        for k in range(VECS):
            sl = pl.ds(k * LANES, LANES)
            ubi_v[j, sl] = lax.shift_right_logical(uid_v[j, sl], 4)
            mbi_v[j, sl] = lax.shift_right_logical(mid_v[j, sl], 4)

    for j in range(NBLK):
        copies.append(pltpu.async_copy(ubt_hbm.at[ubi_v.at[j]], ub_v.at[j], sem))
        copies.append(pltpu.async_copy(mbt_hbm.at[mbi_v.at[j]], mb_v.at[j], sem))
    for c in copies:
        c.wait()

    w0 = w_v[pl.ds(0, LANES)]
    w1 = w_v[pl.ds(LANES, LANES)]
    lanes = lax.iota(jnp.int32, LANES)
    dnums = lax.GatherDimensionNumbers(
        offset_dims=(), collapsed_slice_dims=(0,), start_index_map=(0,))
    wsum = w0 + w1
    for shift in (8, 4, 2, 1):
        rot = (lanes + shift) & (LANES - 1)
        wsum = wsum + lax.gather(
            wsum, rot[:, None], dnums, slice_sizes=(1,),
            mode=lax.GatherScatterMode.PROMISE_IN_BOUNDS)
    outb = outb_v[...]

    for j in range(NBLK):
        def fold_body(r, _, j=j):
            u0 = u_v[j, r, pl.ds(0, LANES)]
            u1 = u_v[j, r, pl.ds(LANES, LANES)]
            m0 = m_v[j, r, pl.ds(0, LANES)]
            m1 = m_v[j, r, pl.ds(LANES, LANES)]
            q_v[j * IDX_BLK + r, :] = u0 * m0 * w0 + u1 * m1 * w1
            return 0
        lax.fori_loop(0, IDX_BLK, fold_body, 0)

    zeros = lanes * 0

    def reduce_body(g, _):
        rows = lanes + g * LANES
        acc = plsc.load_gather(q_v, [rows, zeros])
        for d in range(1, LANES):
            acc = acc + plsc.load_gather(q_v, [rows, zeros + d])
        blk = lax.shift_right_logical(rows, 7)
        r_in = rows & (IDX_BLK - 1)
        uid16 = plsc.load_gather(uid_v, [blk, r_in])
        mid16 = plsc.load_gather(mid_v, [blk, r_in])
        ub = plsc.load_gather(ub_v, [blk, r_in, uid16 & (LANES - 1)])
        mb = plsc.load_gather(mb_v, [blk, r_in, mid16 & (LANES - 1)])
        o_v[pl.ds(g * LANES, LANES)] = acc + wsum * (ub + mb) + outb
        return 0

    lax.fori_loop(0, GROUPS, reduce_body, 0)

    pltpu.sync_copy(o_v, out_hbm.at[pl.ds(base, CHUNK)])


_rec_kernel = pl.kernel(
    _rec_body,
    mesh=plsc.VectorSubcoreMesh(
        core_axis_name="c", subcore_axis_name="s",
        num_cores=NUM_CORES, num_subcores=NUM_SUBCORES),
    out_type=jax.ShapeDtypeStruct((BATCH,), jnp.float32),
    compiler_params=pltpu.CompilerParams(
        needs_layout_passes=False, use_tc_tiling_on_sc=False),
    scratch_types=SCRATCH_TYPES,
)


def kernel(user_ids, movie_tags, user_table, movie_table, user_bias_table,
           movie_bias_table, out_w, out_b):
    uid2d = user_ids.astype(jnp.int32).reshape(NUM_WORKERS * NBLK, IDX_BLK)
    mid2d = movie_tags.astype(jnp.int32).reshape(NUM_WORKERS * NBLK, IDX_BLK)
    ubt2d = user_bias_table.reshape(-1, LANES)
    mbt2d = movie_bias_table.reshape(-1, LANES)
    w_flat = out_w.reshape(EMBED_DIM)
    outb16 = jnp.broadcast_to(out_b, (LANES,))
    out = _rec_kernel(uid2d, mid2d, user_table, movie_table, ubt2d, mbt2d,
                      w_flat, outb16)
    return out.reshape(BATCH, 1)


# speedup vs baseline: 2.7673x; 2.7673x over previous
import jax
import jax.numpy as jnp
from jax import lax
from jax.experimental import pallas as pl
from jax.experimental.pallas import tpu as pltpu
from jax.experimental.pallas import tpu_sc as plsc

NUM_CORES = 2
NUM_SUBCORES = 16
LANES = 16
NUM_WORKERS = NUM_CORES * NUM_SUBCORES

BATCH = 16384
EMBED_DIM = 32
CHUNK = BATCH // NUM_WORKERS
B_R = 32
NBLK = CHUNK // B_R
MAIN_ROWS = 999936
NG = 499968
TAIL = 64
GPB = B_R * EMBED_DIM

SCRATCH_TYPES = [
    pltpu.VMEM((NBLK, B_R), jnp.int32),
    pltpu.VMEM((NBLK, B_R), jnp.int32),
    pltpu.VMEM((NBLK, B_R), jnp.int32),
    pltpu.VMEM((NBLK, B_R), jnp.int32),
    pltpu.VMEM((8, 128), jnp.int32),
    pltpu.VMEM((8, 128), jnp.int32),
    pltpu.VMEM((GPB, LANES), jnp.float32),
    pltpu.VMEM((GPB, LANES), jnp.float32),
    pltpu.VMEM((NBLK, B_R, LANES), jnp.float32),
    pltpu.VMEM((NBLK, B_R, LANES), jnp.float32),
    pltpu.VMEM((TAIL, EMBED_DIM), jnp.float32),
    pltpu.VMEM((TAIL, EMBED_DIM), jnp.float32),
    pltpu.VMEM((EMBED_DIM,), jnp.float32),
    pltpu.VMEM((LANES,), jnp.float32),
    pltpu.VMEM((CHUNK,), jnp.float32),
    pltpu.SemaphoreType.DMA,
    pltpu.SemaphoreType.DMA,
]


def _rec_body(uid_hbm, mid_hbm, ua0, ua1, ua2, ua3, ma0, ma1, ma2, ma3,
              utail_hbm, mtail_hbm, ubt_hbm, mbt_hbm, w_hbm, outb_hbm,
              out_hbm, uid_v, mid_v, ubi_v, mbi_v, ueidx_v, meidx_v, ublk_v,
              mblk_v, ub_v, mb_v, utail_v, mtail_v, w_v, outb_v, o_v,
              sem_e, sem_b):
    uviews = (ua0, ua1, ua2, ua3)
    mviews = (ma0, ma1, ma2, ma3)
    wid = lax.axis_index("s") * NUM_CORES + lax.axis_index("c")
    base = wid * CHUNK

    pltpu.sync_copy(uid_hbm.at[pl.ds(wid * NBLK, NBLK)], uid_v)
    pltpu.sync_copy(mid_hbm.at[pl.ds(wid * NBLK, NBLK)], mid_v)
    pltpu.sync_copy(w_hbm, w_v)
    pltpu.sync_copy(outb_hbm, outb_v)
    pltpu.sync_copy(utail_hbm, utail_v)
    pltpu.sync_copy(mtail_hbm, mtail_v)

    lanes = lax.iota(jnp.int32, LANES)

    for j in range(NBLK):
        for k in range(B_R // LANES):
            sl = pl.ds(k * LANES, LANES)
            ubi_v[j, sl] = lax.shift_right_logical(uid_v[j, sl], 4)
            mbi_v[j, sl] = lax.shift_right_logical(mid_v[j, sl], 4)
    bias_copies = []
    for j in range(NBLK):
        bias_copies.append(
            pltpu.async_copy(ubt_hbm.at[ubi_v.at[j]], ub_v.at[j], sem_b))
        bias_copies.append(
            pltpu.async_copy(mbt_hbm.at[mbi_v.at[j]], mb_v.at[j], sem_b))

    w0 = w_v[pl.ds(0, LANES)]
    w1 = w_v[pl.ds(LANES, LANES)]
    dnums = lax.GatherDimensionNumbers(
        offset_dims=(), collapsed_slice_dims=(0,), start_index_map=(0,))

    def _bcast(vec, lane):
        idx = (lanes * 0 + lane)[:, None]
        return lax.gather(vec, idx, dnums, slice_sizes=(1,),
                          mode=lax.GatherScatterMode.PROMISE_IN_BOUNDS)

    wb = [_bcast(w0, c) if c < LANES else _bcast(w1, c - LANES)
          for c in range(EMBED_DIM)]
    wsum = w0 + w1
    for shift in (8, 4, 2, 1):
        rot = (lanes + shift) & (LANES - 1)
        wsum = wsum + lax.gather(
            wsum, rot[:, None], dnums, slice_sizes=(1,),
            mode=lax.GatherScatterMode.PROMISE_IN_BOUNDS)
    outb = outb_v[...]

    lanes8 = lanes * 8

    def block_body(b, _):
        for grp in range(2):
            usl = uid_v[b, pl.ds(grp * LANES, LANES)]
            msl = mid_v[b, pl.ds(grp * LANES, LANES)]
            ubase = lax.shift_right_logical(usl, 7) * 64 + \
                (lax.shift_right_logical(usl, 4) & 7)
            mbase = lax.shift_right_logical(msl, 7) * 64 + \
                (lax.shift_right_logical(msl, 4) & 7)
            for c8 in range(8):
                uval = jnp.minimum(ubase + c8 * 8, NG - 1)
                mval = jnp.minimum(mbase + c8 * 8, NG - 1)
                for a in range(4):
                    pos = a * 256 + c8 * 32 + grp * LANES
                    ueidx_v[pos // 128, pl.ds(pos % 128, LANES)] = uval
                    meidx_v[pos // 128, pl.ds(pos % 128, LANES)] = mval

        copies = []
        for s in range(8):
            a = s // 2
            dst = pl.ds(s * 128, 128)
            copies.append(pltpu.async_copy(
                uviews[a].at[ueidx_v.at[s]], ublk_v.at[dst], sem_e))
            copies.append(pltpu.async_copy(
                mviews[a].at[meidx_v.at[s]], mblk_v.at[dst], sem_e))
        for c in copies:
            c.wait()

        for grp in range(2):
            usl = uid_v[b, pl.ds(grp * LANES, LANES)]
            msl = mid_v[b, pl.ds(grp * LANES, LANES)]
            ucol = usl & (LANES - 1)
            mcol = msl & (LANES - 1)
            utmask = usl >= MAIN_ROWS
            mtmask = msl >= MAIN_ROWS
            urt = jnp.clip(usl - MAIN_ROWS, 0, TAIL - 1)
            mrt = jnp.clip(msl - MAIN_ROWS, 0, TAIL - 1)
            acc = outb * 0.0
            for c in range(EMBED_DIM):
                a, c8 = divmod(c, 8)
                rows = lanes + (a * 256 + c8 * 32 + grp * LANES)
                uvec = plsc.load_gather(ublk_v, [rows, ucol])
                mvec = plsc.load_gather(mblk_v, [rows, mcol])
                utv = plsc.load_gather(utail_v, [urt, ucol * 0 + c])
                mtv = plsc.load_gather(mtail_v, [mrt, mcol * 0 + c])
                uval = jnp.where(utmask, utv, uvec)
                mval = jnp.where(mtmask, mtv, mvec)
                acc = acc + wb[c] * uval * mval
            o_v[pl.ds(b * B_R + grp * LANES, LANES)] = acc
        return 0

    lax.fori_loop(0, NBLK, block_body, 0)

    for c in bias_copies:
        c.wait()
    for j in range(NBLK):
        for grp in range(2):
            sl = pl.ds(grp * LANES, LANES)
            usl = uid_v[j, sl]
            msl = mid_v[j, sl]
            r_in = lanes + grp * LANES
            ubv = plsc.load_gather(ub_v, [lanes * 0 + j, r_in, usl & 15])
            mbv = plsc.load_gather(mb_v, [lanes * 0 + j, r_in, msl & 15])
            i0 = j * B_R + grp * LANES
            o_v[pl.ds(i0, LANES)] = (
                o_v[pl.ds(i0, LANES)] + wsum * (ubv + mbv) + outb)

    pltpu.sync_copy(o_v, out_hbm.at[pl.ds(base, CHUNK)])


_rec_kernel = pl.kernel(
    _rec_body,
    mesh=plsc.VectorSubcoreMesh(
        core_axis_name="c", subcore_axis_name="s",
        num_cores=NUM_CORES, num_subcores=NUM_SUBCORES),
    out_type=jax.ShapeDtypeStruct((BATCH,), jnp.float32),
    compiler_params=pltpu.CompilerParams(
        needs_layout_passes=False, use_tc_tiling_on_sc=False),
    scratch_types=SCRATCH_TYPES,
)


def _granule_view(table, a):
    return (table[:MAIN_ROWS, 8 * a:8 * a + 8]
            .reshape(MAIN_ROWS // 128, 128, 8)
            .transpose(0, 2, 1)
            .reshape(NG, LANES))


def kernel(user_ids, movie_tags, user_table, movie_table, user_bias_table,
           movie_bias_table, out_w, out_b):
    uid2d = user_ids.astype(jnp.int32).reshape(NUM_WORKERS * NBLK, B_R)
    mid2d = movie_tags.astype(jnp.int32).reshape(NUM_WORKERS * NBLK, B_R)
    uviews = [_granule_view(user_table, a) for a in range(4)]
    mviews = [_granule_view(movie_table, a) for a in range(4)]
    utail = user_table[MAIN_ROWS:]
    mtail = movie_table[MAIN_ROWS:]
    ubt2d = user_bias_table.reshape(-1, LANES)
    mbt2d = movie_bias_table.reshape(-1, LANES)
    w_flat = out_w.reshape(EMBED_DIM)
    outb16 = jnp.broadcast_to(out_b, (LANES,))
    out = _rec_kernel(uid2d, mid2d, *uviews, *mviews, utail, mtail,
                      ubt2d, mbt2d, w_flat, outb16)
    return out.reshape(BATCH, 1)
